# Initial kernel scaffold; baseline (speedup 1.0000x reference)
import jax
import jax.numpy as jnp
from jax.experimental import pallas as pl


def kernel(obs, hidden_states, dones, action, W_ih, W_hh, b_ih, b_hh, W_out, b_out, log_std):
    raise NotImplementedError("write your pallas kernel here")


# speedup vs baseline: 4.6552x; 4.6552x over previous
import functools
import math

import jax
import jax.numpy as jnp
from jax.experimental import pallas as pl
from jax.experimental.pallas import tpu as pltpu

T, B, D, H, A = 2048, 16, 128, 128, 32
TC_CHUNK = 256
NC = T // TC_CHUNK

_HALF_LOG_2PI = 0.5 * math.log(2.0 * math.pi)


def _gru_kernel(obs_ref, dones_ref, act_ref, h0_ref, wih_ref, whh_ref,
                bih_ref, bhh_ref, wout_ref, bout_ref, ls_ref,
                lp_ref, ent_ref, h_s, gi_s, outs_s):
    c = pl.program_id(0)

    @pl.when(c == 0)
    def _init():
        h_s[...] = h0_ref[...]

    x = obs_ref[...].reshape(TC_CHUNK * B, D)
    gi_s[...] = (
        jax.lax.dot_general(x, wih_ref[...], (((1,), (0,)), ((), ())),
                            preferred_element_type=jnp.float32,
                            precision=jax.lax.Precision.HIGHEST)
        + bih_ref[...]
    )

    whh = whh_ref[...]
    bhh = bhh_ref[...]

    def step(t, h):
        dm = dones_ref[t]
        h = h * (1.0 - dm)
        row = pl.multiple_of(t * B, B)
        gi_t = gi_s[pl.ds(row, B), :]
        gh = jax.lax.dot_general(h, whh, (((1,), (0,)), ((), ())),
                                 preferred_element_type=jnp.float32,
                                 precision=jax.lax.Precision.HIGHEST) + bhh
        i_r = gi_t[:, :H]
        i_z = gi_t[:, H:2 * H]
        i_n = gi_t[:, 2 * H:]
        h_r = gh[:, :H]
        h_z = gh[:, H:2 * H]
        h_n = gh[:, 2 * H:]
        r = jax.nn.sigmoid(i_r + h_r)
        z = jax.nn.sigmoid(i_z + h_z)
        n = jnp.tanh(i_n + r * h_n)
        h_new = (1.0 - z) * n + z * h
        outs_s[pl.ds(row, B), :] = h_new
        return h_new

    h_fin = jax.lax.fori_loop(0, TC_CHUNK, step, h_s[...])
    h_s[...] = h_fin

    mean = jax.lax.dot_general(outs_s[...], wout_ref[...],
                               (((1,), (0,)), ((), ())),
                               preferred_element_type=jnp.float32,
                               precision=jax.lax.Precision.HIGHEST) + bout_ref[...]
    a = act_ref[...].reshape(TC_CHUNK * B, A)
    ls = ls_ref[...]
    inv2var = 0.5 * jnp.exp(-2.0 * ls)
    terms = -((a - mean) ** 2) * inv2var - ls - _HALF_LOG_2PI
    lp_ref[...] = jnp.sum(terms, axis=1, keepdims=True)
    ent_ref[...] = jnp.full((TC_CHUNK * B, 1),
                            jnp.sum(0.5 + _HALF_LOG_2PI + ls), jnp.float32)


@jax.jit
def _run(obs, hidden_states, dones, action, W_ih, W_hh, b_ih, b_hh,
         W_out, b_out, log_std):
    obs3 = obs.reshape(T, B, D)
    dones3 = dones.reshape(T, B, 1)
    act3 = action.reshape(T, B, A)
    h0 = hidden_states.reshape(B, H)
    wihT = W_ih.T
    whhT = W_hh.T
    woutT = W_out.T
    bih = b_ih.reshape(1, 3 * H)
    bhh = b_hh.reshape(1, 3 * H)
    bout = b_out.reshape(1, A)
    ls = log_std.reshape(1, A)

    lp, ent = pl.pallas_call(
        _gru_kernel,
        grid=(NC,),
        in_specs=[
            pl.BlockSpec((TC_CHUNK, B, D), lambda c: (c, 0, 0)),
            pl.BlockSpec((TC_CHUNK, B, 1), lambda c: (c, 0, 0)),
            pl.BlockSpec((TC_CHUNK, B, A), lambda c: (c, 0, 0)),
            pl.BlockSpec((B, H), lambda c: (0, 0)),
            pl.BlockSpec((D, 3 * H), lambda c: (0, 0)),
            pl.BlockSpec((H, 3 * H), lambda c: (0, 0)),
            pl.BlockSpec((1, 3 * H), lambda c: (0, 0)),
            pl.BlockSpec((1, 3 * H), lambda c: (0, 0)),
            pl.BlockSpec((H, A), lambda c: (0, 0)),
            pl.BlockSpec((1, A), lambda c: (0, 0)),
            pl.BlockSpec((1, A), lambda c: (0, 0)),
        ],
        out_specs=[
            pl.BlockSpec((TC_CHUNK * B, 1), lambda c: (c, 0)),
            pl.BlockSpec((TC_CHUNK * B, 1), lambda c: (c, 0)),
        ],
        out_shape=[
            jax.ShapeDtypeStruct((T * B, 1), jnp.float32),
            jax.ShapeDtypeStruct((T * B, 1), jnp.float32),
        ],
        scratch_shapes=[
            pltpu.VMEM((B, H), jnp.float32),
            pltpu.VMEM((TC_CHUNK * B, 3 * H), jnp.float32),
            pltpu.VMEM((TC_CHUNK * B, H), jnp.float32),
        ],
        compiler_params=pltpu.CompilerParams(
            dimension_semantics=("arbitrary",)),
    )(obs3, dones3, act3, h0, wihT, whhT, bih, bhh, woutT, bout, ls)

    return action, lp.reshape(T * B), ent.reshape(T * B)


def kernel(obs, hidden_states, dones, action, W_ih, W_hh, b_ih, b_hh,
           W_out, b_out, log_std):
    return _run(obs, hidden_states, dones, action, W_ih, W_hh, b_ih, b_hh,
                W_out, b_out, log_std)


# speedup vs baseline: 13.8708x; 2.9796x over previous
import math

import jax
import jax.numpy as jnp
from jax.experimental import pallas as pl
from jax.experimental.pallas import tpu as pltpu

T, B, D, H, A = 2048, 16, 128, 128, 32
C = 16
L = T // C
CB = C * B
TILE = 64
TB = 256

_HALF_LOG_2PI = 0.5 * math.log(2.0 * math.pi)
_PREC = jax.lax.Precision.HIGHEST


def _scan_kernel(obs_ref, mask_ref, h0_ref, wih_ref, whh_ref, bih_ref,
                 bhh_ref, n_ref, outs_ref, h_all_s):
    h_all_s[...] = jnp.zeros((CB, H), jnp.float32)
    h_all_s[0:B, :] = h0_ref[...]
    wih = wih_ref[...]
    whh = whh_ref[...]
    bih = bih_ref[...]
    bhh = bhh_ref[...]

    def gru_step(x, h, m):
        hm = h * m
        gi = jax.lax.dot_general(x, wih, (((1,), (0,)), ((), ())),
                                 preferred_element_type=jnp.float32,
                                 precision=_PREC) + bih
        gh = jax.lax.dot_general(hm, whh, (((1,), (0,)), ((), ())),
                                 preferred_element_type=jnp.float32,
                                 precision=_PREC) + bhh
        r = jax.nn.sigmoid(gi[:, :H] + gh[:, :H])
        z = jax.nn.sigmoid(gi[:, H:2 * H] + gh[:, H:2 * H])
        n = jnp.tanh(gi[:, 2 * H:] + r * gh[:, 2 * H:])
        return (1.0 - z) * n + z * hm

    def p1_step(s, carry):
        for k in range(CB // TILE):
            ck = k * (TILE // B)
            x = obs_ref[pl.ds(ck, TILE // B), pl.ds(s, 1)].reshape(TILE, D)
            m = mask_ref[pl.ds(ck, TILE // B), pl.ds(s, 1)].reshape(
                TILE, 1).astype(jnp.float32)
            h = h_all_s[pl.ds(k * TILE, TILE), :]
            h_new = gru_step(x, h, m)
            h_all_s[pl.ds(k * TILE, TILE), :] = h_new
            outs_ref[pl.ds(ck, TILE // B), pl.ds(s, 1)] = h_new.reshape(
                TILE // B, 1, B, H).astype(jnp.bfloat16)
        return carry

    jax.lax.fori_loop(0, L, p1_step, 0, unroll=False)

    def chunk_body(c, h):
        nc = n_ref[c]

        def s_body(s, h):
            x = obs_ref[pl.ds(c, 1), pl.ds(s, 1)].reshape(B, D)
            m = mask_ref[pl.ds(c, 1), pl.ds(s, 1)].reshape(
                B, 1).astype(jnp.float32)
            h_new = gru_step(x, h, m)
            outs_ref[pl.ds(c, 1), pl.ds(s, 1)] = h_new.reshape(
                1, 1, B, H).astype(jnp.bfloat16)
            return h_new

        h2 = jax.lax.fori_loop(0, nc, s_body, h)
        row = pl.multiple_of(c * B, B)
        he1 = h_all_s[pl.ds(row, B), :]
        wf = jnp.where(nc == L, 1.0, 0.0).astype(jnp.float32)
        return wf * h2 + (1.0 - wf) * he1

    jax.lax.fori_loop(1, C, chunk_body, h_all_s[0:B, :])


def _head_kernel(outs_ref, act_ref, wout_ref, bout_ref, ls_ref,
                 lp_ref, ent_ref):
    o = outs_ref[...].reshape(TB * B, H).astype(jnp.float32)
    mean = jax.lax.dot_general(o, wout_ref[...], (((1,), (0,)), ((), ())),
                               preferred_element_type=jnp.float32,
                               precision=jax.lax.Precision.HIGHEST) + bout_ref[...]
    a = act_ref[...].reshape(TB * B, A)
    ls = ls_ref[...]
    inv2var = 0.5 * jnp.exp(-2.0 * ls)
    terms = -((a - mean) ** 2) * inv2var - ls - _HALF_LOG_2PI
    lp_ref[...] = jnp.sum(terms, axis=1, keepdims=True)
    ent_ref[...] = jnp.full((TB * B, 1),
                            jnp.sum(0.5 + _HALF_LOG_2PI + ls), jnp.float32)


@jax.jit
def _run(obs, hidden_states, dones, action, W_ih, W_hh, b_ih, b_hh,
         W_out, b_out, log_std):
    obs4 = obs.reshape(C, L, B, D)
    d2 = dones.reshape(C, L, B)
    mask4 = (1.0 - d2).reshape(C, L, B, 1).astype(jnp.bfloat16)
    act3 = action.reshape(T, B, A)
    h0 = hidden_states.reshape(B, H)
    wihT = W_ih.T
    whhT = W_hh.T
    woutT = W_out.T
    bih = b_ih.reshape(1, 3 * H)
    bhh = b_hh.reshape(1, 3 * H)
    bout = b_out.reshape(1, A)
    ls = log_std.reshape(1, A)

    di = (d2 > 0.5)
    first = jnp.argmax(di, axis=1)
    m = jnp.where(di.any(axis=1), first, L)
    n = m.max(axis=1).astype(jnp.int32).at[0].set(0)

    outs4 = pl.pallas_call(
        _scan_kernel,
        grid=(1,),
        in_specs=[
            pl.BlockSpec((C, L, B, D), lambda i: (0, 0, 0, 0)),
            pl.BlockSpec((C, L, B, 1), lambda i: (0, 0, 0, 0)),
            pl.BlockSpec((B, H), lambda i: (0, 0)),
            pl.BlockSpec((D, 3 * H), lambda i: (0, 0)),
            pl.BlockSpec((H, 3 * H), lambda i: (0, 0)),
            pl.BlockSpec((1, 3 * H), lambda i: (0, 0)),
            pl.BlockSpec((1, 3 * H), lambda i: (0, 0)),
            pl.BlockSpec(memory_space=pltpu.SMEM),
        ],
        out_specs=pl.BlockSpec((C, L, B, H), lambda i: (0, 0, 0, 0)),
        out_shape=jax.ShapeDtypeStruct((C, L, B, H), jnp.bfloat16),
        scratch_shapes=[pltpu.VMEM((CB, H), jnp.float32)],
        compiler_params=pltpu.CompilerParams(
            dimension_semantics=("arbitrary",)),
    )(obs4, mask4, h0, wihT, whhT, bih, bhh, n)

    outs3 = outs4.reshape(T, B, H)
    lp, ent = pl.pallas_call(
        _head_kernel,
        grid=(T // TB,),
        in_specs=[
            pl.BlockSpec((TB, B, H), lambda i: (i, 0, 0)),
            pl.BlockSpec((TB, B, A), lambda i: (i, 0, 0)),
            pl.BlockSpec((H, A), lambda i: (0, 0)),
            pl.BlockSpec((1, A), lambda i: (0, 0)),
            pl.BlockSpec((1, A), lambda i: (0, 0)),
        ],
        out_specs=[
            pl.BlockSpec((TB * B, 1), lambda i: (i, 0)),
            pl.BlockSpec((TB * B, 1), lambda i: (i, 0)),
        ],
        out_shape=[
            jax.ShapeDtypeStruct((T * B, 1), jnp.float32),
            jax.ShapeDtypeStruct((T * B, 1), jnp.float32),
        ],
    )(outs3, act3, woutT, bout, ls)

    return action, lp.reshape(T * B), ent.reshape(T * B)


def kernel(obs, hidden_states, dones, action, W_ih, W_hh, b_ih, b_hh,
           W_out, b_out, log_std):
    return _run(obs, hidden_states, dones, action, W_ih, W_hh, b_ih, b_hh,
                W_out, b_out, log_std)


# speedup vs baseline: 22.8963x; 1.6507x over previous
import math

import jax
import jax.numpy as jnp
from jax.experimental import pallas as pl
from jax.experimental.pallas import tpu as pltpu

T, B, D, H, A = 2048, 16, 128, 128, 32
C = 16
L = T // C
CB = C * B
TILE = 64
TB = 256

_HALF_LOG_2PI = 0.5 * math.log(2.0 * math.pi)
_PREC = jax.lax.Precision.DEFAULT


def _scan_kernel(obs_ref, mask_ref, h0_ref, wih_ref, whh_ref, bih_ref,
                 bhh_ref, n_ref, outs_ref, h_all_s):
    h_all_s[...] = jnp.zeros((CB, H), jnp.float32)
    h_all_s[0:B, :] = h0_ref[...]
    wih = wih_ref[...]
    whh = whh_ref[...]
    bih = bih_ref[...]
    bhh = bhh_ref[...]

    def gru_step(x, h, m):
        hm = h * m
        gi = jax.lax.dot_general(x, wih, (((1,), (0,)), ((), ())),
                                 preferred_element_type=jnp.float32,
                                 precision=_PREC) + bih
        gh = jax.lax.dot_general(hm, whh, (((1,), (0,)), ((), ())),
                                 preferred_element_type=jnp.float32,
                                 precision=_PREC) + bhh
        r = jax.nn.sigmoid(gi[:, :H] + gh[:, :H])
        z = jax.nn.sigmoid(gi[:, H:2 * H] + gh[:, H:2 * H])
        n = jnp.tanh(gi[:, 2 * H:] + r * gh[:, 2 * H:])
        return (1.0 - z) * n + z * hm

    def p1_step(s, carry):
        for k in range(CB // TILE):
            ck = k * (TILE // B)
            x = obs_ref[pl.ds(ck, TILE // B), pl.ds(s, 1)].reshape(TILE, D)
            m = mask_ref[pl.ds(ck, TILE // B), pl.ds(s, 1)].reshape(
                TILE, 1).astype(jnp.float32)
            h = h_all_s[pl.ds(k * TILE, TILE), :]
            h_new = gru_step(x, h, m)
            h_all_s[pl.ds(k * TILE, TILE), :] = h_new
            outs_ref[pl.ds(ck, TILE // B), pl.ds(s, 1)] = h_new.reshape(
                TILE // B, 1, B, H).astype(jnp.bfloat16)
        return carry

    jax.lax.fori_loop(0, L, p1_step, 0, unroll=False)

    def chunk_body(c, h):
        nc = n_ref[c]

        def s_body(s, h):
            x = obs_ref[pl.ds(c, 1), pl.ds(s, 1)].reshape(B, D)
            m = mask_ref[pl.ds(c, 1), pl.ds(s, 1)].reshape(
                B, 1).astype(jnp.float32)
            h_new = gru_step(x, h, m)
            outs_ref[pl.ds(c, 1), pl.ds(s, 1)] = h_new.reshape(
                1, 1, B, H).astype(jnp.bfloat16)
            return h_new

        h2 = jax.lax.fori_loop(0, nc, s_body, h)
        row = pl.multiple_of(c * B, B)
        he1 = h_all_s[pl.ds(row, B), :]
        wf = jnp.where(nc == L, 1.0, 0.0).astype(jnp.float32)
        return wf * h2 + (1.0 - wf) * he1

    jax.lax.fori_loop(1, C, chunk_body, h_all_s[0:B, :])


def _head_kernel(outs_ref, act_ref, wout_ref, bout_ref, ls_ref,
                 lp_ref, ent_ref):
    o = outs_ref[...].reshape(TB * B, H).astype(jnp.float32)
    mean = jax.lax.dot_general(o, wout_ref[...], (((1,), (0,)), ((), ())),
                               preferred_element_type=jnp.float32,
                               precision=jax.lax.Precision.HIGHEST) + bout_ref[...]
    a = act_ref[...].reshape(TB * B, A)
    ls = ls_ref[...]
    inv2var = 0.5 * jnp.exp(-2.0 * ls)
    terms = -((a - mean) ** 2) * inv2var - ls - _HALF_LOG_2PI
    lp_ref[...] = jnp.sum(terms, axis=1, keepdims=True)
    ent_ref[...] = jnp.full((TB * B, 1),
                            jnp.sum(0.5 + _HALF_LOG_2PI + ls), jnp.float32)


@jax.jit
def _run(obs, hidden_states, dones, action, W_ih, W_hh, b_ih, b_hh,
         W_out, b_out, log_std):
    obs4 = obs.reshape(C, L, B, D)
    d2 = dones.reshape(C, L, B)
    mask4 = (1.0 - d2).reshape(C, L, B, 1).astype(jnp.bfloat16)
    act3 = action.reshape(T, B, A)
    h0 = hidden_states.reshape(B, H)
    wihT = W_ih.T
    whhT = W_hh.T
    woutT = W_out.T
    bih = b_ih.reshape(1, 3 * H)
    bhh = b_hh.reshape(1, 3 * H)
    bout = b_out.reshape(1, A)
    ls = log_std.reshape(1, A)

    di = (d2 > 0.5)
    first = jnp.argmax(di, axis=1)
    m = jnp.where(di.any(axis=1), first, L)
    n = m.max(axis=1).astype(jnp.int32).at[0].set(0)

    outs4 = pl.pallas_call(
        _scan_kernel,
        grid=(1,),
        in_specs=[
            pl.BlockSpec((C, L, B, D), lambda i: (0, 0, 0, 0)),
            pl.BlockSpec((C, L, B, 1), lambda i: (0, 0, 0, 0)),
            pl.BlockSpec((B, H), lambda i: (0, 0)),
            pl.BlockSpec((D, 3 * H), lambda i: (0, 0)),
            pl.BlockSpec((H, 3 * H), lambda i: (0, 0)),
            pl.BlockSpec((1, 3 * H), lambda i: (0, 0)),
            pl.BlockSpec((1, 3 * H), lambda i: (0, 0)),
            pl.BlockSpec(memory_space=pltpu.SMEM),
        ],
        out_specs=pl.BlockSpec((C, L, B, H), lambda i: (0, 0, 0, 0)),
        out_shape=jax.ShapeDtypeStruct((C, L, B, H), jnp.bfloat16),
        scratch_shapes=[pltpu.VMEM((CB, H), jnp.float32)],
        compiler_params=pltpu.CompilerParams(
            dimension_semantics=("arbitrary",)),
    )(obs4, mask4, h0, wihT, whhT, bih, bhh, n)

    outs3 = outs4.reshape(T, B, H)
    lp, ent = pl.pallas_call(
        _head_kernel,
        grid=(T // TB,),
        in_specs=[
            pl.BlockSpec((TB, B, H), lambda i: (i, 0, 0)),
            pl.BlockSpec((TB, B, A), lambda i: (i, 0, 0)),
            pl.BlockSpec((H, A), lambda i: (0, 0)),
            pl.BlockSpec((1, A), lambda i: (0, 0)),
            pl.BlockSpec((1, A), lambda i: (0, 0)),
        ],
        out_specs=[
            pl.BlockSpec((TB * B, 1), lambda i: (i, 0)),
            pl.BlockSpec((TB * B, 1), lambda i: (i, 0)),
        ],
        out_shape=[
            jax.ShapeDtypeStruct((T * B, 1), jnp.float32),
            jax.ShapeDtypeStruct((T * B, 1), jnp.float32),
        ],
    )(outs3, act3, woutT, bout, ls)

    return action, lp.reshape(T * B), ent.reshape(T * B)


def kernel(obs, hidden_states, dones, action, W_ih, W_hh, b_ih, b_hh,
           W_out, b_out, log_std):
    return _run(obs, hidden_states, dones, action, W_ih, W_hh, b_ih, b_hh,
                W_out, b_out, log_std)


# speedup vs baseline: 24.0553x; 1.0506x over previous
import math

import jax
import jax.numpy as jnp
from jax.experimental import pallas as pl
from jax.experimental.pallas import tpu as pltpu

T, B, D, H, A = 2048, 16, 128, 128, 32
C = 16
L = T // C
CB = C * B
TILE = 64
TB = 256

_HALF_LOG_2PI = 0.5 * math.log(2.0 * math.pi)
_PREC = jax.lax.Precision.DEFAULT


def _scan_kernel(obs_ref, mask_ref, h0_ref, wih_ref, whh_ref, bih_ref,
                 bhh_ref, n_ref, r_ref, nmax_ref, outs_ref, h_all_s, h2a_s):
    h_all_s[...] = jnp.zeros((CB, H), jnp.float32)
    h_all_s[0:B, :] = h0_ref[...]
    wih = wih_ref[...]
    whh = whh_ref[...]
    bih = bih_ref[...]
    bhh = bhh_ref[...]

    def gru_step(x, h, m):
        hm = h * m
        gi = jax.lax.dot_general(x, wih, (((1,), (0,)), ((), ())),
                                 preferred_element_type=jnp.float32,
                                 precision=_PREC) + bih
        gh = jax.lax.dot_general(hm, whh, (((1,), (0,)), ((), ())),
                                 preferred_element_type=jnp.float32,
                                 precision=_PREC) + bhh
        r = jax.nn.sigmoid(gi[:, :H] + gh[:, :H])
        z = jax.nn.sigmoid(gi[:, H:2 * H] + gh[:, H:2 * H])
        n = jnp.tanh(gi[:, 2 * H:] + r * gh[:, 2 * H:])
        return (1.0 - z) * n + z * hm

    def make_batched_step(h_ref):
        def batched_step(s, carry):
            for k in range(CB // TILE):
                ck = k * (TILE // B)
                x = obs_ref[pl.ds(ck, TILE // B), pl.ds(s, 1)].reshape(TILE, D)
                m = mask_ref[pl.ds(ck, TILE // B), pl.ds(s, 1)].reshape(
                    TILE, 1).astype(jnp.float32)
                h = h_ref[pl.ds(k * TILE, TILE), :]
                h_new = gru_step(x, h, m)
                h_ref[pl.ds(k * TILE, TILE), :] = h_new
                outs_ref[pl.ds(ck, TILE // B), pl.ds(s, 1)] = h_new.reshape(
                    TILE // B, 1, B, H).astype(jnp.bfloat16)
            return carry
        return batched_step

    jax.lax.fori_loop(0, L, make_batched_step(h_all_s), 0, unroll=False)

    h2a_s[B:CB, :] = h_all_s[0:CB - B, :]
    h2a_s[0:B, :] = h0_ref[...]
    jax.lax.fori_loop(0, nmax_ref[0], make_batched_step(h2a_s), 0,
                      unroll=False)

    def chunk_body(c, h):
        nc = n_ref[c]
        rc = r_ref[c]

        def s_body(s, h):
            x = obs_ref[pl.ds(c, 1), pl.ds(s, 1)].reshape(B, D)
            m = mask_ref[pl.ds(c, 1), pl.ds(s, 1)].reshape(
                B, 1).astype(jnp.float32)
            h_new = gru_step(x, h, m)
            outs_ref[pl.ds(c, 1), pl.ds(s, 1)] = h_new.reshape(
                1, 1, B, H).astype(jnp.bfloat16)
            return h_new

        h2 = jax.lax.fori_loop(0, rc, s_body, h)
        row = pl.multiple_of(c * B, B)
        he1 = h_all_s[pl.ds(row, B), :]
        h2a_end = h2a_s[pl.ds(row, B), :]
        wb = jnp.where(rc > 0, 1.0, 0.0).astype(jnp.float32)
        wf = jnp.where(nc == L, 1.0, 0.0).astype(jnp.float32)
        h_full = wb * h2 + (1.0 - wb) * h2a_end
        return wf * h_full + (1.0 - wf) * he1

    jax.lax.fori_loop(1, C, chunk_body, h_all_s[0:B, :])


def _head_kernel(outs_ref, act_ref, wout_ref, bout_ref, ls_ref,
                 lp_ref, ent_ref):
    o = outs_ref[...].reshape(TB * B, H).astype(jnp.float32)
    mean = jax.lax.dot_general(o, wout_ref[...], (((1,), (0,)), ((), ())),
                               preferred_element_type=jnp.float32,
                               precision=jax.lax.Precision.HIGHEST) + bout_ref[...]
    a = act_ref[...].reshape(TB * B, A)
    ls = ls_ref[...]
    inv2var = 0.5 * jnp.exp(-2.0 * ls)
    terms = -((a - mean) ** 2) * inv2var - ls - _HALF_LOG_2PI
    lp_ref[...] = jnp.sum(terms, axis=1, keepdims=True)
    ent_ref[...] = jnp.full((TB * B, 1),
                            jnp.sum(0.5 + _HALF_LOG_2PI + ls), jnp.float32)


@jax.jit
def _run(obs, hidden_states, dones, action, W_ih, W_hh, b_ih, b_hh,
         W_out, b_out, log_std):
    obs4 = obs.reshape(C, L, B, D)
    d2 = dones.reshape(C, L, B)
    mask4 = (1.0 - d2).reshape(C, L, B, 1).astype(jnp.bfloat16)
    act3 = action.reshape(T, B, A)
    h0 = hidden_states.reshape(B, H)
    wihT = W_ih.T
    whhT = W_hh.T
    woutT = W_out.T
    bih = b_ih.reshape(1, 3 * H)
    bhh = b_hh.reshape(1, 3 * H)
    bout = b_out.reshape(1, A)
    ls = log_std.reshape(1, A)

    di = (d2 > 0.5)
    first = jnp.argmax(di, axis=1)
    m = jnp.where(di.any(axis=1), first, L)
    n = m.max(axis=1).astype(jnp.int32).at[0].set(0)
    nodone = (m == L).any(axis=1)
    bad = jnp.concatenate([jnp.zeros((1,), jnp.bool_), nodone[:-1]])
    r = jnp.where(bad, n, 0).astype(jnp.int32)
    nmax = jnp.max(n).reshape(1)

    outs4 = pl.pallas_call(
        _scan_kernel,
        grid=(1,),
        in_specs=[
            pl.BlockSpec((C, L, B, D), lambda i: (0, 0, 0, 0)),
            pl.BlockSpec((C, L, B, 1), lambda i: (0, 0, 0, 0)),
            pl.BlockSpec((B, H), lambda i: (0, 0)),
            pl.BlockSpec((D, 3 * H), lambda i: (0, 0)),
            pl.BlockSpec((H, 3 * H), lambda i: (0, 0)),
            pl.BlockSpec((1, 3 * H), lambda i: (0, 0)),
            pl.BlockSpec((1, 3 * H), lambda i: (0, 0)),
            pl.BlockSpec(memory_space=pltpu.SMEM),
            pl.BlockSpec(memory_space=pltpu.SMEM),
            pl.BlockSpec(memory_space=pltpu.SMEM),
        ],
        out_specs=pl.BlockSpec((C, L, B, H), lambda i: (0, 0, 0, 0)),
        out_shape=jax.ShapeDtypeStruct((C, L, B, H), jnp.bfloat16),
        scratch_shapes=[pltpu.VMEM((CB, H), jnp.float32),
                        pltpu.VMEM((CB, H), jnp.float32)],
        compiler_params=pltpu.CompilerParams(
            dimension_semantics=("arbitrary",)),
    )(obs4, mask4, h0, wihT, whhT, bih, bhh, n, r, nmax)

    outs3 = outs4.reshape(T, B, H)
    lp, ent = pl.pallas_call(
        _head_kernel,
        grid=(T // TB,),
        in_specs=[
            pl.BlockSpec((TB, B, H), lambda i: (i, 0, 0)),
            pl.BlockSpec((TB, B, A), lambda i: (i, 0, 0)),
            pl.BlockSpec((H, A), lambda i: (0, 0)),
            pl.BlockSpec((1, A), lambda i: (0, 0)),
            pl.BlockSpec((1, A), lambda i: (0, 0)),
        ],
        out_specs=[
            pl.BlockSpec((TB * B, 1), lambda i: (i, 0)),
            pl.BlockSpec((TB * B, 1), lambda i: (i, 0)),
        ],
        out_shape=[
            jax.ShapeDtypeStruct((T * B, 1), jnp.float32),
            jax.ShapeDtypeStruct((T * B, 1), jnp.float32),
        ],
    )(outs3, act3, woutT, bout, ls)

    return action, lp.reshape(T * B), ent.reshape(T * B)


def kernel(obs, hidden_states, dones, action, W_ih, W_hh, b_ih, b_hh,
           W_out, b_out, log_std):
    return _run(obs, hidden_states, dones, action, W_ih, W_hh, b_ih, b_hh,
                W_out, b_out, log_std)


# speedup vs baseline: 26.2370x; 1.0907x over previous
import math

import jax
import jax.numpy as jnp
from jax.experimental import pallas as pl
from jax.experimental.pallas import tpu as pltpu

T, B, D, H, A = 2048, 16, 128, 128, 32
C = 32
L = T // C
CB = C * B
TILE = 64
TB = 256

_HALF_LOG_2PI = 0.5 * math.log(2.0 * math.pi)
_PREC = jax.lax.Precision.DEFAULT


def _scan_kernel(obs_ref, mask_ref, h0_ref, wih_ref, whh_ref, bih_ref,
                 bhh_ref, n_ref, r_ref, nmax_ref, outs_ref, h_all_s, h2a_s):
    h_all_s[...] = jnp.zeros((CB, H), jnp.float32)
    h_all_s[0:B, :] = h0_ref[...]
    wih = wih_ref[...]
    whh = whh_ref[...]
    bih = bih_ref[...]
    bhh = bhh_ref[...]

    def gru_step(x, h, m):
        hm = h * m
        gi = jax.lax.dot_general(x, wih, (((1,), (0,)), ((), ())),
                                 preferred_element_type=jnp.float32,
                                 precision=_PREC) + bih
        gh = jax.lax.dot_general(hm, whh, (((1,), (0,)), ((), ())),
                                 preferred_element_type=jnp.float32,
                                 precision=_PREC) + bhh
        r = jax.nn.sigmoid(gi[:, :H] + gh[:, :H])
        z = jax.nn.sigmoid(gi[:, H:2 * H] + gh[:, H:2 * H])
        n = jnp.tanh(gi[:, 2 * H:] + r * gh[:, 2 * H:])
        return (1.0 - z) * n + z * hm

    def make_batched_step(h_ref):
        def batched_step(s, carry):
            for k in range(CB // TILE):
                ck = k * (TILE // B)
                x = obs_ref[pl.ds(ck, TILE // B), pl.ds(s, 1)].reshape(TILE, D)
                m = mask_ref[pl.ds(ck, TILE // B), pl.ds(s, 1)].reshape(
                    TILE, 1).astype(jnp.float32)
                h = h_ref[pl.ds(k * TILE, TILE), :]
                h_new = gru_step(x, h, m)
                h_ref[pl.ds(k * TILE, TILE), :] = h_new
                outs_ref[pl.ds(ck, TILE // B), pl.ds(s, 1)] = h_new.reshape(
                    TILE // B, 1, B, H).astype(jnp.bfloat16)
            return carry
        return batched_step

    jax.lax.fori_loop(0, L, make_batched_step(h_all_s), 0, unroll=False)

    h2a_s[B:CB, :] = h_all_s[0:CB - B, :]
    h2a_s[0:B, :] = h0_ref[...]
    jax.lax.fori_loop(0, nmax_ref[0], make_batched_step(h2a_s), 0,
                      unroll=False)

    def chunk_body(c, h):
        nc = n_ref[c]
        rc = r_ref[c]

        def s_body(s, h):
            x = obs_ref[pl.ds(c, 1), pl.ds(s, 1)].reshape(B, D)
            m = mask_ref[pl.ds(c, 1), pl.ds(s, 1)].reshape(
                B, 1).astype(jnp.float32)
            h_new = gru_step(x, h, m)
            outs_ref[pl.ds(c, 1), pl.ds(s, 1)] = h_new.reshape(
                1, 1, B, H).astype(jnp.bfloat16)
            return h_new

        h2 = jax.lax.fori_loop(0, rc, s_body, h)
        row = pl.multiple_of(c * B, B)
        he1 = h_all_s[pl.ds(row, B), :]
        h2a_end = h2a_s[pl.ds(row, B), :]
        wb = jnp.where(rc > 0, 1.0, 0.0).astype(jnp.float32)
        wf = jnp.where(nc == L, 1.0, 0.0).astype(jnp.float32)
        h_full = wb * h2 + (1.0 - wb) * h2a_end
        return wf * h_full + (1.0 - wf) * he1

    jax.lax.fori_loop(1, C, chunk_body, h_all_s[0:B, :])


def _head_kernel(outs_ref, act_ref, wout_ref, bout_ref, ls_ref,
                 lp_ref, ent_ref):
    o = outs_ref[...].reshape(TB * B, H).astype(jnp.float32)
    mean = jax.lax.dot_general(o, wout_ref[...], (((1,), (0,)), ((), ())),
                               preferred_element_type=jnp.float32,
                               precision=jax.lax.Precision.HIGHEST) + bout_ref[...]
    a = act_ref[...].reshape(TB * B, A)
    ls = ls_ref[...]
    inv2var = 0.5 * jnp.exp(-2.0 * ls)
    terms = -((a - mean) ** 2) * inv2var - ls - _HALF_LOG_2PI
    lp_ref[...] = jnp.sum(terms, axis=1, keepdims=True)
    ent_ref[...] = jnp.full((TB * B, 1),
                            jnp.sum(0.5 + _HALF_LOG_2PI + ls), jnp.float32)


@jax.jit
def _run(obs, hidden_states, dones, action, W_ih, W_hh, b_ih, b_hh,
         W_out, b_out, log_std):
    obs4 = obs.reshape(C, L, B, D)
    d2 = dones.reshape(C, L, B)
    mask4 = (1.0 - d2).reshape(C, L, B, 1).astype(jnp.bfloat16)
    act3 = action.reshape(T, B, A)
    h0 = hidden_states.reshape(B, H)
    wihT = W_ih.T
    whhT = W_hh.T
    woutT = W_out.T
    bih = b_ih.reshape(1, 3 * H)
    bhh = b_hh.reshape(1, 3 * H)
    bout = b_out.reshape(1, A)
    ls = log_std.reshape(1, A)

    di = (d2 > 0.5)
    first = jnp.argmax(di, axis=1)
    m = jnp.where(di.any(axis=1), first, L)
    n = m.max(axis=1).astype(jnp.int32).at[0].set(0)
    nodone = (m == L).any(axis=1)
    bad = jnp.concatenate([jnp.zeros((1,), jnp.bool_), nodone[:-1]])
    r = jnp.where(bad, n, 0).astype(jnp.int32)
    nmax = jnp.max(n).reshape(1)

    outs4 = pl.pallas_call(
        _scan_kernel,
        grid=(1,),
        in_specs=[
            pl.BlockSpec((C, L, B, D), lambda i: (0, 0, 0, 0)),
            pl.BlockSpec((C, L, B, 1), lambda i: (0, 0, 0, 0)),
            pl.BlockSpec((B, H), lambda i: (0, 0)),
            pl.BlockSpec((D, 3 * H), lambda i: (0, 0)),
            pl.BlockSpec((H, 3 * H), lambda i: (0, 0)),
            pl.BlockSpec((1, 3 * H), lambda i: (0, 0)),
            pl.BlockSpec((1, 3 * H), lambda i: (0, 0)),
            pl.BlockSpec(memory_space=pltpu.SMEM),
            pl.BlockSpec(memory_space=pltpu.SMEM),
            pl.BlockSpec(memory_space=pltpu.SMEM),
        ],
        out_specs=pl.BlockSpec((C, L, B, H), lambda i: (0, 0, 0, 0)),
        out_shape=jax.ShapeDtypeStruct((C, L, B, H), jnp.bfloat16),
        scratch_shapes=[pltpu.VMEM((CB, H), jnp.float32),
                        pltpu.VMEM((CB, H), jnp.float32)],
        compiler_params=pltpu.CompilerParams(
            dimension_semantics=("arbitrary",)),
    )(obs4, mask4, h0, wihT, whhT, bih, bhh, n, r, nmax)

    outs3 = outs4.reshape(T, B, H)
    lp, ent = pl.pallas_call(
        _head_kernel,
        grid=(T // TB,),
        in_specs=[
            pl.BlockSpec((TB, B, H), lambda i: (i, 0, 0)),
            pl.BlockSpec((TB, B, A), lambda i: (i, 0, 0)),
            pl.BlockSpec((H, A), lambda i: (0, 0)),
            pl.BlockSpec((1, A), lambda i: (0, 0)),
            pl.BlockSpec((1, A), lambda i: (0, 0)),
        ],
        out_specs=[
            pl.BlockSpec((TB * B, 1), lambda i: (i, 0)),
            pl.BlockSpec((TB * B, 1), lambda i: (i, 0)),
        ],
        out_shape=[
            jax.ShapeDtypeStruct((T * B, 1), jnp.float32),
            jax.ShapeDtypeStruct((T * B, 1), jnp.float32),
        ],
    )(outs3, act3, woutT, bout, ls)

    return action, lp.reshape(T * B), ent.reshape(T * B)


def kernel(obs, hidden_states, dones, action, W_ih, W_hh, b_ih, b_hh,
           W_out, b_out, log_std):
    return _run(obs, hidden_states, dones, action, W_ih, W_hh, b_ih, b_hh,
                W_out, b_out, log_std)


# speedup vs baseline: 26.4218x; 1.0070x over previous
import math

import jax
import jax.numpy as jnp
from jax.experimental import pallas as pl
from jax.experimental.pallas import tpu as pltpu

T, B, D, H, A = 2048, 16, 128, 128, 32
C = 64
L = T // C
CB = C * B
TILE = 64
TB = 256

_HALF_LOG_2PI = 0.5 * math.log(2.0 * math.pi)
_PREC = jax.lax.Precision.DEFAULT


def _scan_kernel(obs_ref, mask_ref, h0_ref, wih_ref, whh_ref, bih_ref,
                 bhh_ref, n_ref, r_ref, nmax_ref, outs_ref, h_all_s, h2a_s):
    h_all_s[...] = jnp.zeros((CB, H), jnp.float32)
    h_all_s[0:B, :] = h0_ref[...]
    wih = wih_ref[...]
    whh = whh_ref[...]
    bih = bih_ref[...]
    bhh = bhh_ref[...]

    def gru_step(x, h, m):
        hm = h * m
        gi = jax.lax.dot_general(x, wih, (((1,), (0,)), ((), ())),
                                 preferred_element_type=jnp.float32,
                                 precision=_PREC) + bih
        gh = jax.lax.dot_general(hm, whh, (((1,), (0,)), ((), ())),
                                 preferred_element_type=jnp.float32,
                                 precision=_PREC) + bhh
        r = jax.nn.sigmoid(gi[:, :H] + gh[:, :H])
        z = jax.nn.sigmoid(gi[:, H:2 * H] + gh[:, H:2 * H])
        n = jnp.tanh(gi[:, 2 * H:] + r * gh[:, 2 * H:])
        return (1.0 - z) * n + z * hm

    def make_batched_step(h_ref):
        def batched_step(s, carry):
            for k in range(CB // TILE):
                ck = k * (TILE // B)
                x = obs_ref[pl.ds(ck, TILE // B), pl.ds(s, 1)].reshape(TILE, D)
                m = mask_ref[pl.ds(ck, TILE // B), pl.ds(s, 1)].reshape(
                    TILE, 1).astype(jnp.float32)
                h = h_ref[pl.ds(k * TILE, TILE), :]
                h_new = gru_step(x, h, m)
                h_ref[pl.ds(k * TILE, TILE), :] = h_new
                outs_ref[pl.ds(ck, TILE // B), pl.ds(s, 1)] = h_new.reshape(
                    TILE // B, 1, B, H).astype(jnp.bfloat16)
            return carry
        return batched_step

    jax.lax.fori_loop(0, L, make_batched_step(h_all_s), 0, unroll=False)

    h2a_s[B:CB, :] = h_all_s[0:CB - B, :]
    h2a_s[0:B, :] = h0_ref[...]
    jax.lax.fori_loop(0, nmax_ref[0], make_batched_step(h2a_s), 0,
                      unroll=False)

    def chunk_body(c, h):
        nc = n_ref[c]
        rc = r_ref[c]

        def s_body(s, h):
            x = obs_ref[pl.ds(c, 1), pl.ds(s, 1)].reshape(B, D)
            m = mask_ref[pl.ds(c, 1), pl.ds(s, 1)].reshape(
                B, 1).astype(jnp.float32)
            h_new = gru_step(x, h, m)
            outs_ref[pl.ds(c, 1), pl.ds(s, 1)] = h_new.reshape(
                1, 1, B, H).astype(jnp.bfloat16)
            return h_new

        h2 = jax.lax.fori_loop(0, rc, s_body, h)
        row = pl.multiple_of(c * B, B)
        he1 = h_all_s[pl.ds(row, B), :]
        h2a_end = h2a_s[pl.ds(row, B), :]
        wb = jnp.where(rc > 0, 1.0, 0.0).astype(jnp.float32)
        wf = jnp.where(nc == L, 1.0, 0.0).astype(jnp.float32)
        h_full = wb * h2 + (1.0 - wb) * h2a_end
        return wf * h_full + (1.0 - wf) * he1

    jax.lax.fori_loop(1, C, chunk_body, h_all_s[0:B, :])


def _head_kernel(outs_ref, act_ref, wout_ref, bout_ref, ls_ref,
                 lp_ref, ent_ref):
    o = outs_ref[...].reshape(TB * B, H).astype(jnp.float32)
    mean = jax.lax.dot_general(o, wout_ref[...], (((1,), (0,)), ((), ())),
                               preferred_element_type=jnp.float32,
                               precision=jax.lax.Precision.HIGHEST) + bout_ref[...]
    a = act_ref[...].reshape(TB * B, A)
    ls = ls_ref[...]
    inv2var = 0.5 * jnp.exp(-2.0 * ls)
    terms = -((a - mean) ** 2) * inv2var - ls - _HALF_LOG_2PI
    lp_ref[...] = jnp.sum(terms, axis=1, keepdims=True)
    ent_ref[...] = jnp.full((TB * B, 1),
                            jnp.sum(0.5 + _HALF_LOG_2PI + ls), jnp.float32)


@jax.jit
def _run(obs, hidden_states, dones, action, W_ih, W_hh, b_ih, b_hh,
         W_out, b_out, log_std):
    obs4 = obs.reshape(C, L, B, D)
    d2 = dones.reshape(C, L, B)
    mask4 = (1.0 - d2).reshape(C, L, B, 1).astype(jnp.bfloat16)
    act3 = action.reshape(T, B, A)
    h0 = hidden_states.reshape(B, H)
    wihT = W_ih.T
    whhT = W_hh.T
    woutT = W_out.T
    bih = b_ih.reshape(1, 3 * H)
    bhh = b_hh.reshape(1, 3 * H)
    bout = b_out.reshape(1, A)
    ls = log_std.reshape(1, A)

    di = (d2 > 0.5)
    first = jnp.argmax(di, axis=1)
    m = jnp.where(di.any(axis=1), first, L)
    n = m.max(axis=1).astype(jnp.int32).at[0].set(0)
    nodone = (m == L).any(axis=1)
    bad = jnp.concatenate([jnp.zeros((1,), jnp.bool_), nodone[:-1]])
    r = jnp.where(bad, n, 0).astype(jnp.int32)
    nmax = jnp.max(n).reshape(1)

    outs4 = pl.pallas_call(
        _scan_kernel,
        grid=(1,),
        in_specs=[
            pl.BlockSpec((C, L, B, D), lambda i: (0, 0, 0, 0)),
            pl.BlockSpec((C, L, B, 1), lambda i: (0, 0, 0, 0)),
            pl.BlockSpec((B, H), lambda i: (0, 0)),
            pl.BlockSpec((D, 3 * H), lambda i: (0, 0)),
            pl.BlockSpec((H, 3 * H), lambda i: (0, 0)),
            pl.BlockSpec((1, 3 * H), lambda i: (0, 0)),
            pl.BlockSpec((1, 3 * H), lambda i: (0, 0)),
            pl.BlockSpec(memory_space=pltpu.SMEM),
            pl.BlockSpec(memory_space=pltpu.SMEM),
            pl.BlockSpec(memory_space=pltpu.SMEM),
        ],
        out_specs=pl.BlockSpec((C, L, B, H), lambda i: (0, 0, 0, 0)),
        out_shape=jax.ShapeDtypeStruct((C, L, B, H), jnp.bfloat16),
        scratch_shapes=[pltpu.VMEM((CB, H), jnp.float32),
                        pltpu.VMEM((CB, H), jnp.float32)],
        compiler_params=pltpu.CompilerParams(
            dimension_semantics=("arbitrary",)),
    )(obs4, mask4, h0, wihT, whhT, bih, bhh, n, r, nmax)

    outs3 = outs4.reshape(T, B, H)
    lp, ent = pl.pallas_call(
        _head_kernel,
        grid=(T // TB,),
        in_specs=[
            pl.BlockSpec((TB, B, H), lambda i: (i, 0, 0)),
            pl.BlockSpec((TB, B, A), lambda i: (i, 0, 0)),
            pl.BlockSpec((H, A), lambda i: (0, 0)),
            pl.BlockSpec((1, A), lambda i: (0, 0)),
            pl.BlockSpec((1, A), lambda i: (0, 0)),
        ],
        out_specs=[
            pl.BlockSpec((TB * B, 1), lambda i: (i, 0)),
            pl.BlockSpec((TB * B, 1), lambda i: (i, 0)),
        ],
        out_shape=[
            jax.ShapeDtypeStruct((T * B, 1), jnp.float32),
            jax.ShapeDtypeStruct((T * B, 1), jnp.float32),
        ],
    )(outs3, act3, woutT, bout, ls)

    return action, lp.reshape(T * B), ent.reshape(T * B)


def kernel(obs, hidden_states, dones, action, W_ih, W_hh, b_ih, b_hh,
           W_out, b_out, log_std):
    return _run(obs, hidden_states, dones, action, W_ih, W_hh, b_ih, b_hh,
                W_out, b_out, log_std)


# speedup vs baseline: 27.2631x; 1.0318x over previous
import math

import jax
import jax.numpy as jnp
from jax.experimental import pallas as pl
from jax.experimental.pallas import tpu as pltpu

T, B, D, H, A = 2048, 16, 128, 128, 32
C = 64
L = T // C
CB = C * B
TILE = 64
TB = 256

_HALF_LOG_2PI = 0.5 * math.log(2.0 * math.pi)
_PREC = jax.lax.Precision.DEFAULT


def _scan_kernel(obs_ref, mask_ref, h0_ref, wih_ref, whh_ref, bih_ref,
                 bhh_ref, n_ref, r_ref, nmax_ref, outs_ref, h_all_s, h2a_s):
    h_all_s[...] = jnp.zeros((CB, H), jnp.float32)
    h_all_s[0:B, :] = h0_ref[...]
    wih = wih_ref[...]
    whh = whh_ref[...]
    bih = bih_ref[...]
    bhh = bhh_ref[...]

    def gru_step(x, h, m):
        hm = h * m
        gi = jax.lax.dot_general(x, wih, (((1,), (0,)), ((), ())),
                                 preferred_element_type=jnp.float32,
                                 precision=_PREC) + bih
        gh = jax.lax.dot_general(hm, whh, (((1,), (0,)), ((), ())),
                                 preferred_element_type=jnp.float32,
                                 precision=_PREC) + bhh
        r = 0.5 * jnp.tanh(0.5 * (gi[:, :H] + gh[:, :H])) + 0.5
        z = 0.5 * jnp.tanh(0.5 * (gi[:, H:2 * H] + gh[:, H:2 * H])) + 0.5
        n = jnp.tanh(gi[:, 2 * H:] + r * gh[:, 2 * H:])
        return n + z * (hm - n)

    def make_batched_step(h_ref):
        def batched_step(s, carry):
            for k in range(CB // TILE):
                ck = k * (TILE // B)
                x = obs_ref[pl.ds(ck, TILE // B), pl.ds(s, 1)].reshape(TILE, D)
                m = mask_ref[pl.ds(ck, TILE // B), pl.ds(s, 1)].reshape(
                    TILE, 1).astype(jnp.float32)
                h = h_ref[pl.ds(k * TILE, TILE), :]
                h_new = gru_step(x, h, m)
                h_ref[pl.ds(k * TILE, TILE), :] = h_new
                outs_ref[pl.ds(ck, TILE // B), pl.ds(s, 1)] = h_new.reshape(
                    TILE // B, 1, B, H).astype(jnp.bfloat16)
            return carry
        return batched_step

    jax.lax.fori_loop(0, L, make_batched_step(h_all_s), 0, unroll=False)

    h2a_s[B:CB, :] = h_all_s[0:CB - B, :]
    h2a_s[0:B, :] = h0_ref[...]
    jax.lax.fori_loop(0, nmax_ref[0], make_batched_step(h2a_s), 0,
                      unroll=False)

    def chunk_body(c, h):
        nc = n_ref[c]
        rc = r_ref[c]

        def s_body(s, h):
            x = obs_ref[pl.ds(c, 1), pl.ds(s, 1)].reshape(B, D)
            m = mask_ref[pl.ds(c, 1), pl.ds(s, 1)].reshape(
                B, 1).astype(jnp.float32)
            h_new = gru_step(x, h, m)
            outs_ref[pl.ds(c, 1), pl.ds(s, 1)] = h_new.reshape(
                1, 1, B, H).astype(jnp.bfloat16)
            return h_new

        h2 = jax.lax.fori_loop(0, rc, s_body, h)
        row = pl.multiple_of(c * B, B)
        he1 = h_all_s[pl.ds(row, B), :]
        h2a_end = h2a_s[pl.ds(row, B), :]
        wb = jnp.where(rc > 0, 1.0, 0.0).astype(jnp.float32)
        wf = jnp.where(nc == L, 1.0, 0.0).astype(jnp.float32)
        h_full = wb * h2 + (1.0 - wb) * h2a_end
        return wf * h_full + (1.0 - wf) * he1

    jax.lax.fori_loop(1, C, chunk_body, h_all_s[0:B, :])


def _head_kernel(outs_ref, act_ref, wout_ref, bout_ref, ls_ref,
                 lp_ref, ent_ref):
    o = outs_ref[...].reshape(TB * B, H).astype(jnp.float32)
    mean = jax.lax.dot_general(o, wout_ref[...], (((1,), (0,)), ((), ())),
                               preferred_element_type=jnp.float32,
                               precision=jax.lax.Precision.DEFAULT) + bout_ref[...]
    a = act_ref[...].reshape(TB * B, A)
    ls = ls_ref[...]
    inv2var = 0.5 * jnp.exp(-2.0 * ls)
    terms = -((a - mean) ** 2) * inv2var - ls - _HALF_LOG_2PI
    lp_ref[...] = jnp.sum(terms, axis=1, keepdims=True)
    ent_ref[...] = jnp.full((TB * B, 1),
                            jnp.sum(0.5 + _HALF_LOG_2PI + ls), jnp.float32)


@jax.jit
def _run(obs, hidden_states, dones, action, W_ih, W_hh, b_ih, b_hh,
         W_out, b_out, log_std):
    obs4 = obs.reshape(C, L, B, D)
    d2 = dones.reshape(C, L, B)
    mask4 = (1.0 - d2).reshape(C, L, B, 1).astype(jnp.bfloat16)
    act3 = action.reshape(T, B, A)
    h0 = hidden_states.reshape(B, H)
    wihT = W_ih.T
    whhT = W_hh.T
    woutT = W_out.T
    bih = b_ih.reshape(1, 3 * H)
    bhh = b_hh.reshape(1, 3 * H)
    bout = b_out.reshape(1, A)
    ls = log_std.reshape(1, A)

    di = (d2 > 0.5)
    first = jnp.argmax(di, axis=1)
    m = jnp.where(di.any(axis=1), first, L)
    n = m.max(axis=1).astype(jnp.int32).at[0].set(0)
    nodone = (m == L).any(axis=1)
    bad = jnp.concatenate([jnp.zeros((1,), jnp.bool_), nodone[:-1]])
    r = jnp.where(bad, n, 0).astype(jnp.int32)
    nmax = jnp.max(n).reshape(1)

    outs4 = pl.pallas_call(
        _scan_kernel,
        grid=(1,),
        in_specs=[
            pl.BlockSpec((C, L, B, D), lambda i: (0, 0, 0, 0)),
            pl.BlockSpec((C, L, B, 1), lambda i: (0, 0, 0, 0)),
            pl.BlockSpec((B, H), lambda i: (0, 0)),
            pl.BlockSpec((D, 3 * H), lambda i: (0, 0)),
            pl.BlockSpec((H, 3 * H), lambda i: (0, 0)),
            pl.BlockSpec((1, 3 * H), lambda i: (0, 0)),
            pl.BlockSpec((1, 3 * H), lambda i: (0, 0)),
            pl.BlockSpec(memory_space=pltpu.SMEM),
            pl.BlockSpec(memory_space=pltpu.SMEM),
            pl.BlockSpec(memory_space=pltpu.SMEM),
        ],
        out_specs=pl.BlockSpec((C, L, B, H), lambda i: (0, 0, 0, 0)),
        out_shape=jax.ShapeDtypeStruct((C, L, B, H), jnp.bfloat16),
        scratch_shapes=[pltpu.VMEM((CB, H), jnp.float32),
                        pltpu.VMEM((CB, H), jnp.float32)],
        compiler_params=pltpu.CompilerParams(
            dimension_semantics=("arbitrary",)),
    )(obs4, mask4, h0, wihT, whhT, bih, bhh, n, r, nmax)

    outs3 = outs4.reshape(T, B, H)
    lp, ent = pl.pallas_call(
        _head_kernel,
        grid=(T // TB,),
        in_specs=[
            pl.BlockSpec((TB, B, H), lambda i: (i, 0, 0)),
            pl.BlockSpec((TB, B, A), lambda i: (i, 0, 0)),
            pl.BlockSpec((H, A), lambda i: (0, 0)),
            pl.BlockSpec((1, A), lambda i: (0, 0)),
            pl.BlockSpec((1, A), lambda i: (0, 0)),
        ],
        out_specs=[
            pl.BlockSpec((TB * B, 1), lambda i: (i, 0)),
            pl.BlockSpec((TB * B, 1), lambda i: (i, 0)),
        ],
        out_shape=[
            jax.ShapeDtypeStruct((T * B, 1), jnp.float32),
            jax.ShapeDtypeStruct((T * B, 1), jnp.float32),
        ],
    )(outs3, act3, woutT, bout, ls)

    return action, lp.reshape(T * B), ent.reshape(T * B)


def kernel(obs, hidden_states, dones, action, W_ih, W_hh, b_ih, b_hh,
           W_out, b_out, log_std):
    return _run(obs, hidden_states, dones, action, W_ih, W_hh, b_ih, b_hh,
                W_out, b_out, log_std)
